# Initial kernel scaffold; baseline (speedup 1.0000x reference)
import jax
import jax.numpy as jnp
from jax.experimental import pallas as pl


def kernel(y_pred, y_true, index, u_pos):
    raise NotImplementedError("write your pallas kernel here")


# speedup vs baseline: 1.0356x; 1.0356x over previous
import functools

import jax
import jax.numpy as jnp
from jax import lax
from jax.experimental import pallas as pl
from jax.experimental.pallas import tpu as pltpu
from jax.experimental.pallas import tpu_sc as plsc

B = 4096
BI = 128
NSTEPS = B // BI
MARGIN = 1.0
LAMBDA = 1.0
TAU = 1.0
GAMMA0 = 0.9
GAMMA1 = 0.9

_NC = 2
_NS = 16
_NW = _NC * _NS
_B_PER_W = B // _NW


def _sc_gather(index, table):
    mesh = plsc.VectorSubcoreMesh(core_axis_name="c", subcore_axis_name="s")

    @functools.partial(
        pl.kernel,
        mesh=mesh,
        out_type=jax.ShapeDtypeStruct((B,), jnp.float32),
        scratch_types=[
            pltpu.VMEM((_B_PER_W,), jnp.int32),
            pltpu.VMEM((_B_PER_W,), jnp.float32),
            pltpu.SemaphoreType.DMA,
        ],
    )
    def body(idx_hbm, table_hbm, out_hbm, idx_v, rows_v, sem):
        wid = lax.axis_index("s") * _NC + lax.axis_index("c")
        base = wid * _B_PER_W
        pltpu.sync_copy(idx_hbm.at[pl.ds(base, _B_PER_W)], idx_v)
        pltpu.async_copy(table_hbm.at[idx_v], rows_v, sem).wait()
        pltpu.sync_copy(rows_v, out_hbm.at[pl.ds(base, _B_PER_W)])

    return body(index, table)


def _main_body(yp_r, yp_c, posf_r, posf_c, negf_c, idx_r, idx_c,
               a_out, cnt_out, later_out, t_out,
               cnt_scr, later_scr, t_scr):
    k = pl.program_id(0)

    @pl.when(k == 0)
    def _init():
        cnt_scr[...] = jnp.zeros_like(cnt_scr)
        later_scr[...] = jnp.zeros_like(later_scr)
        t_scr[0] = 0.0

    d = yp_r[...] - yp_c[...]
    th = jnp.maximum(MARGIN - d, 0.0)
    s = th * th
    e = jnp.exp(s * (1.0 / LAMBDA))
    a_out[...] = jnp.sum(e * negf_c[...], axis=1, keepdims=True)
    t_scr[0] += jnp.sum((e * s) * posf_r[...] * negf_c[...])

    eqf = jnp.where(idx_r[...] == idx_c[...], 1.0, 0.0) * posf_r[...]
    cnt_scr[...] += jnp.sum(eqf, axis=0, keepdims=True)
    row_gid = k * BI + lax.broadcasted_iota(jnp.int32, (BI, 1), 0)
    col_gid = lax.broadcasted_iota(jnp.int32, (1, B), 1)
    later_scr[...] += jnp.sum(
        jnp.where(row_gid > col_gid, eqf, 0.0), axis=0, keepdims=True)

    @pl.when(k == NSTEPS - 1)
    def _fin():
        cnt_out[...] = cnt_scr[...]
        later_out[...] = later_scr[...]
        t_out[...] = jnp.broadcast_to(t_scr[0], (1, 1))


def _epi_body(a, u_g, cnt, later, t, posf, negf, out):
    n_neg = jnp.sum(negf[...])
    u_new = (1.0 - GAMMA0) * u_g[...] + GAMMA0 * (a[...] / n_neg)
    cntw = cnt[...] * jnp.where(later[...] == 0.0, 1.0, 0.0) * posf[...]
    wsum = jnp.dot(cntw, u_new, preferred_element_type=jnp.float32)
    out[...] = t[...] / (GAMMA1 * wsum * n_neg)


def kernel(y_pred, y_true, index, u_pos):
    yp_c = y_pred.reshape(1, B).astype(jnp.float32)
    yp_r = yp_c.reshape(B, 1)
    posf_c = (y_true == 1).astype(jnp.float32).reshape(1, B)
    posf_r = posf_c.reshape(B, 1)
    negf_c = (y_true == 0).astype(jnp.float32).reshape(1, B)
    idx_c = index.reshape(1, B)
    idx_r = index.reshape(B, 1)

    u_g = _sc_gather(index.reshape(B), u_pos.reshape(-1))

    row_spec = pl.BlockSpec((BI, 1), lambda k: (k, 0))
    full_c = pl.BlockSpec((1, B), lambda k: (0, 0))
    a, cnt, later, t = pl.pallas_call(
        _main_body,
        grid=(NSTEPS,),
        in_specs=[row_spec, full_c, row_spec, full_c, full_c, row_spec,
                  full_c],
        out_specs=[pl.BlockSpec((BI, 1), lambda k: (k, 0)), full_c, full_c,
                   pl.BlockSpec((1, 1), lambda k: (0, 0))],
        out_shape=[
            jax.ShapeDtypeStruct((B, 1), jnp.float32),
            jax.ShapeDtypeStruct((1, B), jnp.float32),
            jax.ShapeDtypeStruct((1, B), jnp.float32),
            jax.ShapeDtypeStruct((1, 1), jnp.float32),
        ],
        scratch_shapes=[
            pltpu.VMEM((1, B), jnp.float32),
            pltpu.VMEM((1, B), jnp.float32),
            pltpu.SMEM((1,), jnp.float32),
        ],
    )(yp_r, yp_c, posf_r, posf_c, negf_c, idx_r, idx_c)

    loss = pl.pallas_call(
        _epi_body,
        out_shape=jax.ShapeDtypeStruct((1, 1), jnp.float32),
    )(a, u_g.reshape(B, 1), cnt, later, t, posf_c, negf_c)
    return loss[0, 0]


# speedup vs baseline: 1.2676x; 1.2240x over previous
import functools

import jax
import jax.numpy as jnp
from jax import lax
from jax.experimental import pallas as pl
from jax.experimental.pallas import tpu as pltpu
from jax.experimental.pallas import tpu_sc as plsc

B = 4096
BI = 256
NSTEPS = B // BI
MARGIN = 1.0
LAMBDA = 1.0
TAU = 1.0
GAMMA0 = 0.9
GAMMA1 = 0.9

_NC = 2
_NS = 16
_NW = _NC * _NS
_B_PER_W = B // _NW
_D = 128


def _sc_gather(row_idx, table):
    mesh = plsc.VectorSubcoreMesh(core_axis_name="c", subcore_axis_name="s",
                                  num_cores=_NC, num_subcores=_NS)

    @functools.partial(
        pl.kernel,
        mesh=mesh,
        out_type=jax.ShapeDtypeStruct((B, _D), jnp.float32),
        scratch_types=[
            pltpu.VMEM((_B_PER_W,), jnp.int32),
            pltpu.VMEM((_B_PER_W, _D), jnp.float32),
            pltpu.SemaphoreType.DMA,
        ],
    )
    def body(idx_hbm, table_hbm, out_hbm, idx_v, rows_v, sem):
        wid = lax.axis_index("s") * _NC + lax.axis_index("c")
        base = wid * _B_PER_W
        pltpu.sync_copy(idx_hbm.at[pl.ds(base, _B_PER_W)], idx_v)
        pltpu.async_copy(table_hbm.at[idx_v], rows_v, sem).wait()
        pltpu.sync_copy(rows_v, out_hbm.at[pl.ds(base, _B_PER_W)])

    return body(row_idx, table)


_DN_RHS_T = (((1,), (1,)), ((), ()))


def _main_body(yp_c, yp_bT, posf_row, negf_c, idx_c, idx_bT,
               a_out, cnt_out, later_out, t_out,
               cnt_scr, later_scr, t_scr):
    k = pl.program_id(0)

    @pl.when(k == 0)
    def _init():
        cnt_scr[...] = jnp.zeros_like(cnt_scr)
        later_scr[...] = jnp.zeros_like(later_scr)
        t_scr[0] = 0.0

    yp_r = jnp.transpose(yp_bT[...], (1, 0))
    z = (MARGIN - yp_r) + yp_c[...]
    th = jnp.maximum(z, 0.0)
    s = th * th
    e = jnp.exp(s * (1.0 / LAMBDA))
    es = e * s
    negr = negf_c[...]
    a_out[...] = lax.dot_general(e, negr, _DN_RHS_T,
                                 preferred_element_type=jnp.float32)
    trow = lax.dot_general(es, negr, _DN_RHS_T,
                           preferred_element_type=jnp.float32)
    prow = posf_row[...]
    t_scr[0] += jnp.dot(prow, trow,
                        preferred_element_type=jnp.float32)[0, 0]

    idx_r = jnp.transpose(idx_bT[...], (1, 0))
    eqf = jnp.where(idx_r == idx_c[...], 1.0, 0.0)
    cnt_scr[...] += jnp.dot(prow, eqf, preferred_element_type=jnp.float32)
    row_gid = k * BI + lax.broadcasted_iota(jnp.int32, (BI, 1), 0)
    col_gid = lax.broadcasted_iota(jnp.int32, (1, B), 1)
    laterf = jnp.where(row_gid > col_gid, eqf, 0.0)
    later_scr[...] += jnp.dot(prow, laterf, preferred_element_type=jnp.float32)

    @pl.when(k == NSTEPS - 1)
    def _fin():
        cnt_out[...] = cnt_scr[...]
        later_out[...] = later_scr[...]
        t_out[...] = jnp.broadcast_to(t_scr[0], (1, 1))


def _epi_body(a, u_rows, idx_b, cnt, later, t, posf, negf, out):
    n_neg = jnp.sum(negf[...])
    lane = jnp.bitwise_and(idx_b[...], _D - 1)
    onehot = jnp.where(
        lax.broadcasted_iota(jnp.int32, (1, _D), 1) == lane, 1.0, 0.0)
    u_gc = jnp.sum(u_rows[...] * onehot, axis=1, keepdims=True)
    u_new = (1.0 - GAMMA0) * u_gc + GAMMA0 * (a[...] / n_neg)
    cntw = cnt[...] * jnp.where(later[...] == 0.0, 1.0, 0.0) * posf[...]
    wsum = jnp.dot(cntw, u_new, preferred_element_type=jnp.float32)
    out[...] = t[...] / (GAMMA1 * wsum * n_neg)


def kernel(y_pred, y_true, index, u_pos):
    yp_c = y_pred.reshape(1, B).astype(jnp.float32)
    posf_c = (y_true == 1).astype(jnp.float32).reshape(1, B)
    negf_c = (y_true == 0).astype(jnp.float32).reshape(1, B)
    idx_c = index.reshape(1, B)

    u_flat = jnp.pad(u_pos.reshape(-1), (0, 7813 * _D - 1000000))
    u_rows = _sc_gather(lax.shift_right_logical(index.reshape(B), 7),
                        u_flat.reshape(-1, _D))

    rowT_spec = pl.BlockSpec((1, BI), lambda k: (0, k))
    full_c = pl.BlockSpec((1, B), lambda k: (0, 0))
    a, cnt, later, t = pl.pallas_call(
        _main_body,
        grid=(NSTEPS,),
        in_specs=[full_c, rowT_spec, rowT_spec, full_c, full_c, rowT_spec],
        out_specs=[pl.BlockSpec((BI, 1), lambda k: (k, 0)), full_c, full_c,
                   pl.BlockSpec((1, 1), lambda k: (0, 0))],
        out_shape=[
            jax.ShapeDtypeStruct((B, 1), jnp.float32),
            jax.ShapeDtypeStruct((1, B), jnp.float32),
            jax.ShapeDtypeStruct((1, B), jnp.float32),
            jax.ShapeDtypeStruct((1, 1), jnp.float32),
        ],
        scratch_shapes=[
            pltpu.VMEM((1, B), jnp.float32),
            pltpu.VMEM((1, B), jnp.float32),
            pltpu.SMEM((1,), jnp.float32),
        ],
    )(yp_c, yp_c, posf_c, negf_c, idx_c, idx_c)

    loss = pl.pallas_call(
        _epi_body,
        out_shape=jax.ShapeDtypeStruct((1, 1), jnp.float32),
    )(a, u_rows, index.reshape(B, 1), cnt, later, t, posf_c, negf_c)
    return loss[0, 0]
